# baseline (speedup 1.0000x reference)
import functools

import jax
import jax.numpy as jnp
from jax import lax
from jax.experimental import pallas as pl
from jax.experimental.pallas import tpu as pltpu
from jax.experimental.pallas import tpu_sc as plsc

BATCH = 4096
HIST = 200
EMBED_DIM = 64

NC = 2
NS = 16
NW = NC * NS
TOTAL = BATCH * HIST
PER_W = TOTAL // NW
CHUNK = 128
NCHUNK = PER_W // CHUNK


def _gather_body(words_hbm, table_hbm, out_hbm, idx_v, rows_v, sem):
    wid = lax.axis_index("s") * NC + lax.axis_index("c")
    base = wid * PER_W
    pltpu.sync_copy(words_hbm.at[wid], idx_v)

    def chunk(j, _):
        pltpu.async_copy(table_hbm.at[idx_v.at[j]], rows_v, sem).wait()
        pltpu.sync_copy(rows_v, out_hbm.at[pl.ds(base + j * CHUNK, CHUNK)])
        return ()

    lax.fori_loop(0, NCHUNK, chunk, (), unroll=False)


@jax.jit
def kernel(words, table):
    mesh = plsc.VectorSubcoreMesh(core_axis_name="c", subcore_axis_name="s")
    words_r = words.reshape(NW, NCHUNK, CHUNK)
    out = pl.kernel(
        _gather_body,
        out_type=jax.ShapeDtypeStruct((TOTAL, EMBED_DIM), jnp.float32),
        mesh=mesh,
        scratch_types=[
            pltpu.VMEM((NCHUNK, CHUNK), jnp.int32),
            pltpu.VMEM((CHUNK, EMBED_DIM), jnp.float32),
            pltpu.SemaphoreType.DMA,
        ],
        compiler_params=pltpu.CompilerParams(use_tc_tiling_on_sc=False),
    )(words_r, table)
    return out.reshape(BATCH, HIST, EMBED_DIM)


# speedup vs baseline: 1.1128x; 1.1128x over previous
import jax
import jax.numpy as jnp
from jax import lax
from jax.experimental import pallas as pl
from jax.experimental.pallas import tpu as pltpu
from jax.experimental.pallas import tpu_sc as plsc

BATCH = 4096
HIST = 200
EMBED_DIM = 64

NC = 2
NS = 16
NW = NC * NS
TOTAL = BATCH * HIST
PER_W = TOTAL // NW
CHUNK = 128
NCHUNK = PER_W // CHUNK
K = 5
GROUP = K * CHUNK
NGRP = NCHUNK // K


def _gather_body(words_hbm, table_hbm, out_hbm, idx_v, rows_v, gsems, osems):
    wid = lax.axis_index("s") * NC + lax.axis_index("c")
    base = wid * PER_W
    pltpu.sync_copy(words_hbm.at[wid], idx_v)

    def fire_gathers(g, h):
        for k in range(K):
            pltpu.async_copy(
                table_hbm.at[idx_v.at[g * K + k]],
                rows_v.at[h, pl.ds(k * CHUNK, CHUNK)],
                gsems[h],
            )

    def drain_gathers(h):
        pltpu.make_async_copy(
            table_hbm.at[pl.ds(0, GROUP)], rows_v.at[h], gsems[h]
        ).wait()

    def fire_write(g, h):
        pltpu.async_copy(
            rows_v.at[h], out_hbm.at[pl.ds(base + g * GROUP, GROUP)], osems[h]
        )

    def drain_write(h):
        pltpu.make_async_copy(
            rows_v.at[h], out_hbm.at[pl.ds(base, GROUP)], osems[h]
        ).wait()

    fire_gathers(0, 0)

    def body(t, _):
        for h in (0, 1):
            g = 2 * t + h
            @pl.when(jnp.logical_and(g >= 1, g + 1 < NGRP))
            def _():
                drain_write(1 - h)

            @pl.when(g + 1 < NGRP)
            def _():
                fire_gathers(g + 1, 1 - h)

            drain_gathers(h)
            fire_write(g, h)
        return ()

    lax.fori_loop(0, NGRP // 2, body, (), unroll=False)
    drain_write(0)
    drain_write(1)


@jax.jit
def kernel(words, table):
    mesh = plsc.VectorSubcoreMesh(core_axis_name="c", subcore_axis_name="s")
    words_r = words.reshape(NW, NCHUNK, CHUNK)
    out = pl.kernel(
        _gather_body,
        out_type=jax.ShapeDtypeStruct((TOTAL, EMBED_DIM), jnp.float32),
        mesh=mesh,
        scratch_types=[
            pltpu.VMEM((NCHUNK, CHUNK), jnp.int32),
            pltpu.VMEM((2, GROUP, EMBED_DIM), jnp.float32),
            [pltpu.SemaphoreType.DMA, pltpu.SemaphoreType.DMA],
            [pltpu.SemaphoreType.DMA, pltpu.SemaphoreType.DMA],
        ],
        compiler_params=pltpu.CompilerParams(use_tc_tiling_on_sc=False),
    )(words_r, table)
    return out.reshape(BATCH, HIST, EMBED_DIM)


# speedup vs baseline: 1.1173x; 1.0040x over previous
import jax
import jax.numpy as jnp
from jax import lax
from jax.experimental import pallas as pl
from jax.experimental.pallas import tpu as pltpu
from jax.experimental.pallas import tpu_sc as plsc

BATCH = 4096
HIST = 200
EMBED_DIM = 64

NC = 2
NS = 16
NW = NC * NS
TOTAL = BATCH * HIST
PER_W = TOTAL // NW
GROUP = 640
NGRP = PER_W // GROUP


def _gather_body(words_hbm, table_hbm, out_hbm, idx_v, rows_v, gsems, osems):
    wid = lax.axis_index("s") * NC + lax.axis_index("c")
    base = wid * PER_W
    pltpu.sync_copy(words_hbm.at[wid], idx_v)

    def fire_gather(g, h):
        pltpu.async_copy(
            table_hbm.at[idx_v.at[g]],
            rows_v.at[h],
            gsems[h],
        )

    def drain_gather(h):
        pltpu.make_async_copy(
            table_hbm.at[pl.ds(0, GROUP)], rows_v.at[h], gsems[h]
        ).wait()

    def fire_write(g, h):
        pltpu.async_copy(
            rows_v.at[h], out_hbm.at[pl.ds(base + g * GROUP, GROUP)], osems[h]
        )

    def drain_write(h):
        pltpu.make_async_copy(
            rows_v.at[h], out_hbm.at[pl.ds(base, GROUP)], osems[h]
        ).wait()

    fire_gather(0, 0)

    def body(t, _):
        for h in (0, 1):
            g = 2 * t + h
            @pl.when(jnp.logical_and(g >= 1, g + 1 < NGRP))
            def _():
                drain_write(1 - h)

            @pl.when(g + 1 < NGRP)
            def _():
                fire_gather(g + 1, 1 - h)

            drain_gather(h)
            fire_write(g, h)
        return ()

    lax.fori_loop(0, NGRP // 2, body, (), unroll=False)
    drain_write(0)
    drain_write(1)


@jax.jit
def kernel(words, table):
    mesh = plsc.VectorSubcoreMesh(core_axis_name="c", subcore_axis_name="s")
    words_r = words.reshape(NW, NGRP, GROUP)
    out = pl.kernel(
        _gather_body,
        out_type=jax.ShapeDtypeStruct((TOTAL, EMBED_DIM), jnp.float32),
        mesh=mesh,
        scratch_types=[
            pltpu.VMEM((NGRP, GROUP), jnp.int32),
            pltpu.VMEM((2, GROUP, EMBED_DIM), jnp.float32),
            [pltpu.SemaphoreType.DMA, pltpu.SemaphoreType.DMA],
            [pltpu.SemaphoreType.DMA, pltpu.SemaphoreType.DMA],
        ],
        compiler_params=pltpu.CompilerParams(use_tc_tiling_on_sc=False),
    )(words_r, table)
    return out.reshape(BATCH, HIST, EMBED_DIM)
